# baseline (speedup 1.0000x reference)
import math

import jax
import jax.numpy as jnp
from jax.experimental import pallas as pl
from jax.experimental.pallas import tpu as pltpu

B, T, V = 16, 150, 25
TV = T * V
EPS = 1e-5
NEG = 0.05
NSTAT = float(B * T * V)

CHUNK_T = 5
L = CHUNK_T * V
LP = 128
NCH = T // CHUNK_T
TVP = NCH * LP

F32 = jnp.float32
BF16 = jnp.bfloat16


def _dot(a, b, dims, out=F32):
    return jax.lax.dot_general(a, b, (dims, ((), ())),
                               preferred_element_type=out)


def _time_mean(xf):
    chunks = [xf[:, k * LP:(k + 1) * LP] for k in range(NCH)]
    while len(chunks) > 1:
        nxt = [chunks[i] + chunks[i + 1]
               for i in range(0, len(chunks) - 1, 2)]
        if len(chunks) % 2:
            nxt.append(chunks[-1])
        chunks = nxt
    c = chunks[0]
    d = (c[:, 0:25] + c[:, 25:50] + c[:, 50:75] + c[:, 75:100]
         + c[:, 100:125])
    return d * (1.0 / T)


def _adjacency(xf, cin):
    e = _time_mean(xf)
    logits = _dot(e, e, ((0,), (0,))) * (1.0 / math.sqrt(float(cin)))
    m = jnp.max(logits, axis=-1, keepdims=True)
    ex = jnp.exp(logits - m)
    return ex / jnp.sum(ex, axis=-1, keepdims=True)


def _mix_mat(adj):
    i = jax.lax.broadcasted_iota(jnp.int32, (LP, V), 0)
    k = jax.lax.broadcasted_iota(jnp.int32, (LP, V), 1)
    s = jnp.where((i % V == k) & (i < L), 1.0, 0.0).astype(F32)
    sa = _dot(s, adj, ((1,), (0,)))
    m0 = _dot(sa, s, ((1,), (1,)))
    r = jax.lax.broadcasted_iota(jnp.int32, (LP, LP), 0) // V
    c = jax.lax.broadcasted_iota(jnp.int32, (LP, LP), 1) // V
    return jnp.where(r == c, m0, 0.0)


def _layer(xf, w, b, cin, store, want_stats):
    adj = _adjacency(xf, cin)
    h = _dot(w.astype(BF16), xf.astype(BF16), ((1,), (0,))) + b
    hb = h.astype(BF16)
    mix = _mix_mat(adj).astype(BF16)
    cout = hb.shape[0]
    acc_s = jnp.zeros((cout, LP), dtype=F32)
    acc_q = jnp.zeros((cout, LP), dtype=F32)
    for kk in range(NCH):
        yc = _dot(hb[:, kk * LP:(kk + 1) * LP], mix, ((1,), (0,)))
        store(kk, yc)
        if want_stats:
            acc_s = acc_s + yc
            acc_q = acc_q + yc * yc
    if want_stats:
        s = jnp.sum(acc_s, axis=1, keepdims=True)
        q = jnp.sum(acc_q, axis=1, keepdims=True)
        return jnp.concatenate([s, q], axis=1)
    return None


def _bn_leaky(y, tot, g, be):
    mean = tot[:, 0:1] / NSTAT
    var = tot[:, 1:2] / NSTAT - mean * mean
    inv = jax.lax.rsqrt(var + EPS)
    xh = (y - mean) * inv * g + be
    return jnp.maximum(xh, NEG * xh)


def _fused(x_ref, w1_ref, b1_ref, g1_ref, be1_ref, w2_ref, b2_ref, g2_ref,
           be2_ref, w3_ref, b3_ref, y_ref, y1_s, y2_s, st1_s, st2_s):
    sid = pl.program_id(0)
    bid = pl.program_id(1)

    def _acc(st_ref, new):
        old = st_ref[...]
        st_ref[...] = jnp.where(bid == 0, new, old + new)

    @pl.when(sid == 0)
    def _stage0():
        def store(kk, yc):
            y1_s[pl.ds(bid * 128, 128), pl.ds(kk * LP, LP)] = yc
        st = _layer(x_ref[0], w1_ref[...], b1_ref[...], 193, store, True)
        _acc(st1_s, st)

    @pl.when(sid == 1)
    def _stage1():
        y1 = y1_s[pl.ds(bid * 128, 128), :]
        x2 = _bn_leaky(y1, st1_s[...], g1_ref[...], be1_ref[...])

        def store(kk, yc):
            y2_s[pl.ds(bid * 64, 64), pl.ds(kk * LP, LP)] = yc
        st = _layer(x2, w2_ref[...], b2_ref[...], 128, store, True)
        _acc(st2_s, st)

    @pl.when(sid == 2)
    def _stage2():
        y2 = y2_s[pl.ds(bid * 64, 64), :]
        x3 = _bn_leaky(y2, st2_s[...], g2_ref[...], be2_ref[...])

        def store(kk, yc):
            y_ref[0, :, kk * L:(kk + 1) * L] = yc[:, :L]
        _layer(x3, w3_ref[...], b3_ref[...], 64, store, False)


def _full(shape):
    return pl.BlockSpec(shape, lambda s, b: (0,) * len(shape))


def kernel(x, W1, b1, g1, be1, W2, b2, g2, be2, W3, b3):
    xp = jnp.pad(x.reshape(B, 193, NCH, L),
                 ((0, 0), (0, 0), (0, 0), (0, LP - L))).reshape(B, 193, TVP)

    y3 = pl.pallas_call(
        _fused,
        grid=(3, B),
        in_specs=[
            pl.BlockSpec((1, 193, TVP), lambda s, b: (jnp.where(s == 0, b, 0),
                                                      0, 0)),
            _full((128, 193)), _full((128, 1)), _full((128, 1)),
            _full((128, 1)), _full((64, 128)), _full((64, 1)),
            _full((64, 1)), _full((64, 1)), _full((3, 64)), _full((3, 1)),
        ],
        out_specs=pl.BlockSpec((1, 3, TV), lambda s, b: (b, 0, 0)),
        out_shape=jax.ShapeDtypeStruct((B, 3, TV), F32),
        scratch_shapes=[
            pltpu.VMEM((B * 128, TVP), F32),
            pltpu.VMEM((B * 64, TVP), F32),
            pltpu.VMEM((128, 2), F32),
            pltpu.VMEM((64, 2), F32),
        ],
    )(xp, W1, b1.reshape(128, 1), g1.reshape(128, 1), be1.reshape(128, 1),
      W2, b2.reshape(64, 1), g2.reshape(64, 1), be2.reshape(64, 1),
      W3, b3.reshape(3, 1))

    return y3.reshape(B, 3, T, V)


# speedup vs baseline: 1.6503x; 1.6503x over previous
import math

import jax
import jax.numpy as jnp
from jax.experimental import pallas as pl
from jax.experimental.pallas import tpu as pltpu

B, T, V = 16, 150, 25
TV = T * V
EPS = 1e-5
NEG = 0.05
NSTAT = float(B * T * V)

CHUNK_T = 5
L = CHUNK_T * V
NCH = T // CHUNK_T

F32 = jnp.float32
BF16 = jnp.bfloat16


def _dot(a, b, dims):
    return jax.lax.dot_general(a, b, (dims, ((), ())),
                               preferred_element_type=F32)


def _time_mean(xf):
    a = xf[:, :1875] + xf[:, 1875:]
    b = a[:, :625] + a[:, 625:1250] + a[:, 1250:]
    c = b[:, :125] + b[:, 125:250] + b[:, 250:375] + b[:, 375:500] + b[:, 500:]
    d = c[:, :25] + c[:, 25:50] + c[:, 50:75] + c[:, 75:100] + c[:, 100:]
    return d * (1.0 / T)


def _adjacency(xf, cin):
    e = _time_mean(xf)
    logits = _dot(e, e, ((0,), (0,))) * (1.0 / math.sqrt(float(cin)))
    m = jnp.max(logits, axis=-1, keepdims=True)
    ex = jnp.exp(logits - m)
    return ex / jnp.sum(ex, axis=-1, keepdims=True)


def _mix_mat(adj):
    i = jax.lax.broadcasted_iota(jnp.int32, (L, V), 0)
    k = jax.lax.broadcasted_iota(jnp.int32, (L, V), 1)
    s = jnp.where(i % V == k, 1.0, 0.0).astype(F32)
    sa = _dot(s, adj, ((1,), (0,)))
    m0 = _dot(sa, s, ((1,), (1,)))
    r = jax.lax.broadcasted_iota(jnp.int32, (L, L), 0) // V
    c = jax.lax.broadcasted_iota(jnp.int32, (L, L), 1) // V
    return jnp.where(r == c, m0, 0.0)


def _layer(xf, w, b, cin, store, want_stats):
    adj = _adjacency(xf, cin)
    h = _dot(w.astype(BF16), xf.astype(BF16), ((1,), (0,))) + b
    hb = h.astype(BF16)
    mix = _mix_mat(adj).astype(BF16)
    cout = h.shape[0]
    acc_s = jnp.zeros((cout, L), dtype=F32)
    acc_q = jnp.zeros((cout, L), dtype=F32)
    for kk in range(NCH):
        yc = _dot(hb[:, kk * L:(kk + 1) * L], mix, ((1,), (0,)))
        store(kk, yc)
        if want_stats:
            acc_s = acc_s + yc
            acc_q = acc_q + yc * yc
    if want_stats:
        s = jnp.sum(acc_s, axis=1, keepdims=True)
        q = jnp.sum(acc_q, axis=1, keepdims=True)
        return jnp.concatenate([s, q], axis=1)
    return None


def _bn_leaky(y, tot, g, be):
    mean = tot[:, 0:1] / NSTAT
    var = tot[:, 1:2] / NSTAT - mean * mean
    inv = jax.lax.rsqrt(var + EPS)
    xh = (y - mean) * inv * g + be
    return jnp.maximum(xh, NEG * xh)


def _fused(x_ref, w1_ref, b1_ref, g1_ref, be1_ref, w2_ref, b2_ref, g2_ref,
           be2_ref, w3_ref, b3_ref, y_ref, y1_s, y2_s, st1_s, st2_s):
    sid = pl.program_id(0)
    bid = pl.program_id(1)

    def _acc(st_ref, new):
        old = st_ref[...]
        st_ref[...] = jnp.where(bid == 0, new, old + new)

    @pl.when(sid == 0)
    def _stage0():
        def store(kk, yc):
            y1_s[pl.ds(bid * 128, 128), pl.ds(kk * L, L)] = yc
        st = _layer(x_ref[0], w1_ref[...], b1_ref[...], 193, store, True)
        _acc(st1_s, st)

    @pl.when(sid == 1)
    def _stage1():
        y1 = y1_s[pl.ds(bid * 128, 128), :]
        x2 = _bn_leaky(y1, st1_s[...], g1_ref[...], be1_ref[...])

        def store(kk, yc):
            y2_s[pl.ds(bid * 64, 64), pl.ds(kk * L, L)] = yc
        st = _layer(x2, w2_ref[...], b2_ref[...], 128, store, True)
        _acc(st2_s, st)

    @pl.when(sid == 2)
    def _stage2():
        y2 = y2_s[pl.ds(bid * 64, 64), :]
        x3 = _bn_leaky(y2, st2_s[...], g2_ref[...], be2_ref[...])

        def store(kk, yc):
            y_ref[0, :, kk * L:(kk + 1) * L] = yc
        _layer(x3, w3_ref[...], b3_ref[...], 64, store, False)


def _full(shape):
    return pl.BlockSpec(shape, lambda s, b: (0,) * len(shape))


def kernel(x, W1, b1, g1, be1, W2, b2, g2, be2, W3, b3):
    x2d = x.reshape(B, 193, TV)

    y3 = pl.pallas_call(
        _fused,
        grid=(3, B),
        in_specs=[
            pl.BlockSpec((1, 193, TV), lambda s, b: (jnp.where(s == 0, b, 0),
                                                     0, 0)),
            _full((128, 193)), _full((128, 1)), _full((128, 1)),
            _full((128, 1)), _full((64, 128)), _full((64, 1)),
            _full((64, 1)), _full((64, 1)), _full((3, 64)), _full((3, 1)),
        ],
        out_specs=pl.BlockSpec((1, 3, TV), lambda s, b: (b, 0, 0)),
        out_shape=jax.ShapeDtypeStruct((B, 3, TV), F32),
        scratch_shapes=[
            pltpu.VMEM((B * 128, TV), F32),
            pltpu.VMEM((B * 64, TV), F32),
            pltpu.VMEM((128, 2), F32),
            pltpu.VMEM((64, 2), F32),
        ],
    )(x2d, W1, b1.reshape(128, 1), g1.reshape(128, 1), be1.reshape(128, 1),
      W2, b2.reshape(64, 1), g2.reshape(64, 1), be2.reshape(64, 1),
      W3, b3.reshape(3, 1))

    return y3.reshape(B, 3, T, V)
